# initial kernel scaffold (unmeasured)
import jax
import jax.numpy as jnp
from jax import lax
from jax.experimental import pallas as pl
from jax.experimental.pallas import tpu as pltpu

M_OUT = 512
F = 4096


def kernel(x, dy):
    def body(x_ref, dy_ref, out_ref, send_buf, recv_buf, send_sem, recv_sem):
        my_x = lax.axis_index("x")
        my_y = lax.axis_index("y")
        my_z = lax.axis_index("z")
        partner = (my_x, 1 - my_y, my_z)

        barrier_sem = pltpu.get_barrier_semaphore()
        pl.semaphore_signal(
            barrier_sem, inc=1, device_id=partner,
            device_id_type=pl.DeviceIdType.MESH,
        )
        pl.semaphore_wait(barrier_sem, 1)

        def contrib(col_start):
            return lax.dot_general(
                x_ref[:, pl.ds(col_start, M_OUT)],
                dy_ref[...],
                dimension_numbers=(((0,), (0,)), ((), ())),
                preferred_element_type=jnp.float32,
            )

        send_buf[...] = contrib((1 - my_y) * M_OUT).astype(jnp.bfloat16)
        rdma = pltpu.make_async_remote_copy(
            src_ref=send_buf,
            dst_ref=recv_buf,
            send_sem=send_sem,
            recv_sem=recv_sem,
            device_id=partner,
            device_id_type=pl.DeviceIdType.MESH,
        )
        rdma.start()

        mine = contrib(my_y * M_OUT)
        rdma.wait()
        out_ref[...] = mine + recv_buf[...].astype(jnp.float32)

    return pl.pallas_call(
        body,
        out_shape=jax.ShapeDtypeStruct((M_OUT, F), jnp.float32),
        in_specs=[
            pl.BlockSpec(memory_space=pltpu.VMEM),
            pl.BlockSpec(memory_space=pltpu.VMEM),
        ],
        out_specs=pl.BlockSpec(memory_space=pltpu.VMEM),
        scratch_shapes=[
            pltpu.VMEM((M_OUT, F), jnp.bfloat16),
            pltpu.VMEM((M_OUT, F), jnp.bfloat16),
            pltpu.SemaphoreType.DMA,
            pltpu.SemaphoreType.DMA,
        ],
        compiler_params=pltpu.CompilerParams(collective_id=0),
    )(x, dy)


# baseline (device time: 75077 ns/iter reference)
import jax
import jax.numpy as jnp
from jax import lax
from jax.experimental import pallas as pl
from jax.experimental.pallas import tpu as pltpu

M_OUT = 512
F = 4096


def kernel(x, dy):
    def body(x_ref, dy_ref, out_ref, send_buf, recv_buf, send_sem, recv_sem):
        my_x = lax.axis_index("x")
        my_y = lax.axis_index("y")
        my_z = lax.axis_index("z")
        partner = (my_x, 1 - my_y, my_z)

        barrier_sem = pltpu.get_barrier_semaphore()
        pl.semaphore_signal(
            barrier_sem, inc=1, device_id=partner,
            device_id_type=pl.DeviceIdType.MESH,
        )
        pl.semaphore_wait(barrier_sem, 1)

        def contrib(col_start):
            return lax.dot_general(
                x_ref[:, pl.ds(col_start, M_OUT)],
                dy_ref[...],
                dimension_numbers=(((0,), (0,)), ((), ())),
                preferred_element_type=jnp.float32,
            )

        send_buf[...] = contrib((1 - my_y) * M_OUT).astype(jnp.bfloat16)
        rdma = pltpu.make_async_remote_copy(
            src_ref=send_buf,
            dst_ref=recv_buf,
            send_sem=send_sem,
            recv_sem=recv_sem,
            device_id=partner,
            device_id_type=pl.DeviceIdType.MESH,
        )
        rdma.start()

        mine = contrib(my_y * M_OUT)
        rdma.wait()
        out_ref[...] = mine + recv_buf[...].astype(jnp.float32)

    return pl.pallas_call(
        body,
        out_shape=jax.ShapeDtypeStruct((M_OUT, F), jnp.float32),
        in_specs=[
            pl.BlockSpec(memory_space=pltpu.VMEM),
            pl.BlockSpec(memory_space=pltpu.VMEM),
        ],
        out_specs=pl.BlockSpec(memory_space=pltpu.VMEM),
        scratch_shapes=[
            pltpu.VMEM((M_OUT, F), jnp.bfloat16),
            pltpu.VMEM((M_OUT, F), jnp.bfloat16),
            pltpu.SemaphoreType.DMA,
            pltpu.SemaphoreType.DMA,
        ],
        compiler_params=pltpu.CompilerParams(
            collective_id=0, vmem_limit_bytes=60 * 1024 * 1024
        ),
    )(x, dy)


# device time: 50829 ns/iter; 1.4771x vs baseline; 1.4771x over previous
import jax
import jax.numpy as jnp
from jax import lax
from jax.experimental import pallas as pl
from jax.experimental.pallas import tpu as pltpu

M_OUT = 512
F = 4096
P = 128
C = 4
FC = F // C

_MESH = pl.DeviceIdType.MESH


def kernel(x, dy):
    def body(
        x_ref, dy_ref, out_ref,
        bsend, brecv, mypiece, ag,
        ysend_sem, yrecv_sem,
        xsend_sem, xrecv_sem,
        z1send_sem, z1recv_sem,
        z2send_sem, z2recv_sem,
    ):
        mx = lax.axis_index("x")
        my = lax.axis_index("y")
        mz = lax.axis_index("z")
        p = mx * 2 + mz
        q = (1 - mx) * 2 + mz
        r = mx * 2 + (1 - mz)
        s = (1 - mx) * 2 + (1 - mz)
        ypart = (mx, 1 - my, mz)
        xnbr = (1 - mx, my, mz)
        znbr = (mx, my, 1 - mz)

        barrier_sem = pltpu.get_barrier_semaphore()
        for nbr in (ypart, xnbr, znbr):
            pl.semaphore_signal(barrier_sem, inc=1, device_id=nbr,
                                device_id_type=_MESH)
        pl.semaphore_wait(barrier_sem, 3)

        def contrib(y_sel, c):
            return lax.dot_general(
                x_ref[:, pl.ds(y_sel * M_OUT + p * P, P)],
                dy_ref[:, pl.ds(c * FC, FC)],
                dimension_numbers=(((0,), (0,)), ((), ())),
                preferred_element_type=jnp.float32,
            )

        y_rdmas = []
        for c in range(C):
            cs = pl.ds(c * FC, FC)
            bsend[:, cs] = contrib(1 - my, c).astype(jnp.bfloat16)
            rd = pltpu.make_async_remote_copy(
                src_ref=bsend.at[:, cs], dst_ref=brecv.at[:, cs],
                send_sem=ysend_sem.at[c], recv_sem=yrecv_sem.at[c],
                device_id=ypart, device_id_type=_MESH)
            rd.start()
            y_rdmas.append(rd)

        x_rdmas, z1_rdmas = [], []
        for c in range(C):
            cs = pl.ds(c * FC, FC)
            a_c = contrib(my, c)
            y_rdmas[c].wait_recv()
            sum_c = a_c + brecv[:, cs].astype(jnp.float32)
            out_ref[pl.ds(p * P, P), cs] = sum_c
            mypiece[:, cs] = sum_c.astype(jnp.bfloat16)
            rd_x = pltpu.make_async_remote_copy(
                src_ref=mypiece.at[:, cs], dst_ref=ag.at[p, :, cs],
                send_sem=xsend_sem.at[c], recv_sem=xrecv_sem.at[c],
                device_id=xnbr, device_id_type=_MESH)
            rd_x.start()
            x_rdmas.append(rd_x)
            rd_z1 = pltpu.make_async_remote_copy(
                src_ref=mypiece.at[:, cs], dst_ref=ag.at[p, :, cs],
                send_sem=z1send_sem.at[c], recv_sem=z1recv_sem.at[c],
                device_id=znbr, device_id_type=_MESH)
            rd_z1.start()
            z1_rdmas.append(rd_z1)

        z2_rdmas = []
        for c in range(C):
            cs = pl.ds(c * FC, FC)
            recv_x = pltpu.make_async_remote_copy(
                src_ref=mypiece.at[:, cs], dst_ref=ag.at[q, :, cs],
                send_sem=xsend_sem.at[c], recv_sem=xrecv_sem.at[c],
                device_id=xnbr, device_id_type=_MESH)
            recv_x.wait_recv()
            rd_z2 = pltpu.make_async_remote_copy(
                src_ref=ag.at[q, :, cs], dst_ref=ag.at[q, :, cs],
                send_sem=z2send_sem.at[c], recv_sem=z2recv_sem.at[c],
                device_id=znbr, device_id_type=_MESH)
            rd_z2.start()
            z2_rdmas.append(rd_z2)

        for c in range(C):
            cs = pl.ds(c * FC, FC)
            recv_z1 = pltpu.make_async_remote_copy(
                src_ref=mypiece.at[:, cs], dst_ref=ag.at[r, :, cs],
                send_sem=z1send_sem.at[c], recv_sem=z1recv_sem.at[c],
                device_id=znbr, device_id_type=_MESH)
            recv_z1.wait_recv()
            recv_z2 = pltpu.make_async_remote_copy(
                src_ref=mypiece.at[:, cs], dst_ref=ag.at[s, :, cs],
                send_sem=z2send_sem.at[c], recv_sem=z2recv_sem.at[c],
                device_id=znbr, device_id_type=_MESH)
            recv_z2.wait_recv()

        for slot in (q, r, s):
            out_ref[pl.ds(slot * P, P), :] = ag[slot, :, :].astype(jnp.float32)

        for rd in y_rdmas + x_rdmas + z1_rdmas + z2_rdmas:
            rd.wait_send()

    return pl.pallas_call(
        body,
        out_shape=jax.ShapeDtypeStruct((M_OUT, F), jnp.float32),
        in_specs=[
            pl.BlockSpec(memory_space=pltpu.VMEM),
            pl.BlockSpec(memory_space=pltpu.VMEM),
        ],
        out_specs=pl.BlockSpec(memory_space=pltpu.VMEM),
        scratch_shapes=[
            pltpu.VMEM((P, F), jnp.bfloat16),
            pltpu.VMEM((P, F), jnp.bfloat16),
            pltpu.VMEM((P, F), jnp.bfloat16),
            pltpu.VMEM((4, P, F), jnp.bfloat16),
            pltpu.SemaphoreType.DMA((C,)),
            pltpu.SemaphoreType.DMA((C,)),
            pltpu.SemaphoreType.DMA((C,)),
            pltpu.SemaphoreType.DMA((C,)),
            pltpu.SemaphoreType.DMA((C,)),
            pltpu.SemaphoreType.DMA((C,)),
            pltpu.SemaphoreType.DMA((C,)),
            pltpu.SemaphoreType.DMA((C,)),
        ],
        compiler_params=pltpu.CompilerParams(
            collective_id=0, vmem_limit_bytes=60 * 1024 * 1024
        ),
    )(x, dy)
